# baseline (speedup 1.0000x reference)
import functools

import jax
import jax.numpy as jnp
from jax import lax
from jax.experimental import pallas as pl
from jax.experimental.pallas import tpu as pltpu
from jax.experimental.pallas import tpu_sc as plsc

K = 8192
C = 32
T = 1024
KCH = 2048
NKCH = K // KCH


def _vq_codes_kernel(z_ref, cb_ref, codes_ref):
    zb = z_ref[0]
    z2 = jnp.sum(zb * zb, axis=0, keepdims=True)
    cbv = cb_ref[...]
    cn2_full = jnp.sum(cbv * cbv, axis=1, keepdims=True)
    kio = lax.broadcasted_iota(jnp.int32, (KCH, T), 0).astype(jnp.float32)
    best = jnp.full((1, T), jnp.inf, dtype=jnp.float32)
    bestidx = jnp.zeros((1, T), dtype=jnp.float32)
    for kc in range(NKCH):
        cbm2 = -2.0 * cb_ref[kc * KCH:(kc + 1) * KCH, :]
        t1 = z2 + cn2_full[kc * KCH:(kc + 1) * KCH, :]
        s = lax.dot_general(cbm2, zb, (((1,), (0,)), ((), ())),
                            preferred_element_type=jnp.float32)
        d = t1 + s
        cmin = jnp.min(d, axis=0, keepdims=True)
        carg = jnp.min(jnp.where(d == cmin, kio, float(K)), axis=0,
                       keepdims=True)
        upd = cmin < best
        best = jnp.where(upd, cmin, best)
        bestidx = jnp.where(upd, carg + float(kc * KCH), bestidx)
    codes_ref[0] = bestidx.astype(jnp.int32)


def _epilogue_kernel(q_ref, z_ref, out_ref, loss_ref):
    q = q_ref[0][:, :C]
    zb = z_ref[0]
    qt = jnp.transpose(q, (1, 0))
    diff = qt - zb
    out_ref[0] = zb + diff
    loss_ref[...] = jnp.sum(diff * diff)[None, None, None]


def _sc_info():
    try:
        info = plsc.get_sparse_core_info()
        return info.num_cores, info.num_subcores
    except Exception:
        return 2, 16


def kernel(z, codebook):
    B = z.shape[0]
    BT = B * T

    codes3 = pl.pallas_call(
        _vq_codes_kernel,
        grid=(B,),
        in_specs=[
            pl.BlockSpec((1, C, T), lambda b: (b, 0, 0)),
            pl.BlockSpec((K, C), lambda b: (0, 0)),
        ],
        out_specs=pl.BlockSpec((1, 1, T), lambda b: (b, 0, 0)),
        out_shape=jax.ShapeDtypeStruct((B, 1, T), jnp.int32),
        compiler_params=pltpu.CompilerParams(
            dimension_semantics=("parallel",)),
    )(z, codebook)
    codes = codes3.reshape(B, T)

    nc, ns = _sc_info()
    nw = nc * ns
    bpw = BT // nw

    cb_pad = jnp.pad(codebook, ((0, 0), (0, 128 - C)))

    @functools.partial(
        pl.kernel,
        mesh=plsc.VectorSubcoreMesh(core_axis_name="c", subcore_axis_name="s"),
        out_type=jax.ShapeDtypeStruct((BT, 128), jnp.float32),
        scratch_types=[
            pltpu.VMEM((bpw,), jnp.int32),
            pltpu.VMEM((bpw, 128), jnp.float32),
            pltpu.SemaphoreType.DMA,
        ],
    )
    def _sc_gather(cb_hbm, idx_hbm, out_hbm, idx_v, rows_v, sem):
        wid = lax.axis_index("s") * nc + lax.axis_index("c")
        base = wid * bpw
        pltpu.sync_copy(idx_hbm.at[pl.ds(base, bpw)], idx_v)
        pltpu.async_copy(cb_hbm.at[idx_v], rows_v, sem).wait()
        pltpu.sync_copy(rows_v, out_hbm.at[pl.ds(base, bpw)])

    qrows = _sc_gather(cb_pad, codes.reshape(BT))

    quant, loss_parts = pl.pallas_call(
        _epilogue_kernel,
        grid=(B,),
        in_specs=[
            pl.BlockSpec((1, T, 128), lambda b: (b, 0, 0)),
            pl.BlockSpec((1, C, T), lambda b: (b, 0, 0)),
        ],
        out_specs=[
            pl.BlockSpec((1, C, T), lambda b: (b, 0, 0)),
            pl.BlockSpec((1, 1, 1), lambda b: (b, 0, 0)),
        ],
        out_shape=[
            jax.ShapeDtypeStruct((B, C, T), jnp.float32),
            jax.ShapeDtypeStruct((B, 1, 1), jnp.float32),
        ],
        compiler_params=pltpu.CompilerParams(
            dimension_semantics=("parallel",)),
    )(qrows.reshape(B, T, 128), z)

    commit_loss = jnp.sum(loss_parts) / (B * C * T)
    return codes, quant, commit_loss


# speedup vs baseline: 1.0331x; 1.0331x over previous
import functools

import jax
import jax.numpy as jnp
from jax import lax
from jax.experimental import pallas as pl
from jax.experimental.pallas import tpu as pltpu
from jax.experimental.pallas import tpu_sc as plsc

K = 8192
C = 32
T = 1024
KCH = 1024
NKCH = K // KCH


def _vq_codes_kernel(z_ref, cb_ref, codes_ref):
    zb = z_ref[0]
    z2 = jnp.sum(zb * zb, axis=0, keepdims=True)
    cbv = cb_ref[...]
    cn2_full = jnp.sum(cbv * cbv, axis=1, keepdims=True)
    kio = lax.broadcasted_iota(jnp.int32, (KCH, T), 0).astype(jnp.float32)
    best = jnp.full((1, T), jnp.inf, dtype=jnp.float32)
    bestidx = jnp.zeros((1, T), dtype=jnp.float32)
    for kc in range(NKCH):
        cbm2 = -2.0 * cb_ref[kc * KCH:(kc + 1) * KCH, :]
        t1 = z2 + cn2_full[kc * KCH:(kc + 1) * KCH, :]
        s = lax.dot_general(cbm2, zb, (((1,), (0,)), ((), ())),
                            preferred_element_type=jnp.float32)
        d = t1 + s
        cmin = jnp.min(d, axis=0, keepdims=True)
        carg = jnp.min(jnp.where(d == cmin, kio, float(K)), axis=0,
                       keepdims=True)
        upd = cmin < best
        best = jnp.where(upd, cmin, best)
        bestidx = jnp.where(upd, carg + float(kc * KCH), bestidx)
    codes_ref[0] = bestidx.astype(jnp.int32)


def _epilogue_kernel(q_ref, z_ref, out_ref, loss_ref):
    q = q_ref[0]
    zb = z_ref[0]
    qt = jnp.transpose(q, (1, 0))
    diff = qt - zb
    out_ref[0] = zb + diff
    loss_ref[...] = jnp.sum(diff * diff)[None, None, None]


def _sc_info():
    try:
        info = plsc.get_sparse_core_info()
        return info.num_cores, info.num_subcores
    except Exception:
        return 2, 16


def kernel(z, codebook):
    B = z.shape[0]
    BT = B * T

    codes3 = pl.pallas_call(
        _vq_codes_kernel,
        grid=(B,),
        in_specs=[
            pl.BlockSpec((1, C, T), lambda b: (b, 0, 0)),
            pl.BlockSpec((K, C), lambda b: (0, 0)),
        ],
        out_specs=pl.BlockSpec((1, 1, T), lambda b: (b, 0, 0)),
        out_shape=jax.ShapeDtypeStruct((B, 1, T), jnp.int32),
        compiler_params=pltpu.CompilerParams(
            dimension_semantics=("parallel",)),
    )(z, codebook)
    codes = codes3.reshape(B, T)

    nc, ns = _sc_info()
    nw = nc * ns
    bpw = BT // nw

    @functools.partial(
        pl.kernel,
        mesh=plsc.VectorSubcoreMesh(core_axis_name="c", subcore_axis_name="s"),
        out_type=jax.ShapeDtypeStruct((BT, C), jnp.float32),
        scratch_types=[
            pltpu.VMEM((bpw,), jnp.int32),
            pltpu.VMEM((bpw, C), jnp.float32),
            pltpu.SemaphoreType.DMA,
        ],
        compiler_params=pltpu.CompilerParams(use_tc_tiling_on_sc=False),
    )
    def _sc_gather(cb_hbm, idx_hbm, out_hbm, idx_v, rows_v, sem):
        wid = lax.axis_index("s") * nc + lax.axis_index("c")
        base = wid * bpw
        pltpu.sync_copy(idx_hbm.at[pl.ds(base, bpw)], idx_v)
        pltpu.async_copy(cb_hbm.at[idx_v], rows_v, sem).wait()
        pltpu.sync_copy(rows_v, out_hbm.at[pl.ds(base, bpw)])

    qrows = _sc_gather(codebook, codes.reshape(BT))

    quant, loss_parts = pl.pallas_call(
        _epilogue_kernel,
        grid=(B,),
        in_specs=[
            pl.BlockSpec((1, T, C), lambda b: (b, 0, 0)),
            pl.BlockSpec((1, C, T), lambda b: (b, 0, 0)),
        ],
        out_specs=[
            pl.BlockSpec((1, C, T), lambda b: (b, 0, 0)),
            pl.BlockSpec((1, 1, 1), lambda b: (b, 0, 0)),
        ],
        out_shape=[
            jax.ShapeDtypeStruct((B, C, T), jnp.float32),
            jax.ShapeDtypeStruct((B, 1, 1), jnp.float32),
        ],
        compiler_params=pltpu.CompilerParams(
            dimension_semantics=("parallel",)),
    )(qrows.reshape(B, T, C), z)

    commit_loss = jnp.sum(loss_parts) / (B * C * T)
    return codes, quant, commit_loss


# speedup vs baseline: 1.0756x; 1.0411x over previous
import functools

import jax
import jax.numpy as jnp
from jax import lax
from jax.experimental import pallas as pl
from jax.experimental.pallas import tpu as pltpu
from jax.experimental.pallas import tpu_sc as plsc

K = 8192
C = 32
T = 1024
KCH = 1024
NKCH = K // KCH


def _vq_codes_kernel(z_ref, cb_ref, codes_ref):
    zb = z_ref[0]
    z2 = jnp.sum(zb * zb, axis=0, keepdims=True)
    cbv = cb_ref[...]
    cn2_full = jnp.sum(cbv * cbv, axis=1, keepdims=True)
    kio = lax.broadcasted_iota(jnp.int32, (KCH, T), 0).astype(jnp.float32)
    best = jnp.full((1, T), jnp.inf, dtype=jnp.float32)
    bestidx = jnp.zeros((1, T), dtype=jnp.float32)
    for kc in range(NKCH):
        cbm2 = -2.0 * cb_ref[kc * KCH:(kc + 1) * KCH, :]
        t1 = z2 + cn2_full[kc * KCH:(kc + 1) * KCH, :]
        s = lax.dot_general(cbm2, zb, (((1,), (0,)), ((), ())),
                            preferred_element_type=jnp.float32)
        d = t1 + s
        cmin = jnp.min(d, axis=0, keepdims=True)
        carg = jnp.min(jnp.where(d == cmin, kio, float(K)), axis=0,
                       keepdims=True)
        upd = cmin < best
        best = jnp.where(upd, cmin, best)
        bestidx = jnp.where(upd, carg + float(kc * KCH), bestidx)
    codes_ref[0] = bestidx.astype(jnp.int32)


def _epilogue_kernel(q_ref, z_ref, out_ref, loss_ref):
    qt = jnp.transpose(q_ref[...], (0, 2, 1))
    zb = z_ref[...]
    diff = qt - zb
    out_ref[...] = zb + diff
    loss_ref[...] = jnp.sum(diff * diff)[None, None]


def _sc_info():
    try:
        info = plsc.get_sparse_core_info()
        return info.num_cores, info.num_subcores
    except Exception:
        return 2, 16


def kernel(z, codebook):
    B = z.shape[0]
    BT = B * T

    codes3 = pl.pallas_call(
        _vq_codes_kernel,
        grid=(B,),
        in_specs=[
            pl.BlockSpec((1, C, T), lambda b: (b, 0, 0)),
            pl.BlockSpec((K, C), lambda b: (0, 0)),
        ],
        out_specs=pl.BlockSpec((1, 1, T), lambda b: (b, 0, 0)),
        out_shape=jax.ShapeDtypeStruct((B, 1, T), jnp.int32),
        compiler_params=pltpu.CompilerParams(
            dimension_semantics=("parallel",)),
    )(z, codebook)
    codes = codes3.reshape(B, T)

    nc, ns = _sc_info()
    nw = nc * ns
    bpw = BT // nw

    @functools.partial(
        pl.kernel,
        mesh=plsc.VectorSubcoreMesh(core_axis_name="c", subcore_axis_name="s"),
        out_type=jax.ShapeDtypeStruct((BT, C), jnp.float32),
        scratch_types=[
            pltpu.VMEM((bpw,), jnp.int32),
            pltpu.VMEM((bpw, C), jnp.float32),
            pltpu.SemaphoreType.DMA,
        ],
        compiler_params=pltpu.CompilerParams(use_tc_tiling_on_sc=False),
    )
    def _sc_gather(cb_hbm, idx_hbm, out_hbm, idx_v, rows_v, sem):
        wid = lax.axis_index("s") * nc + lax.axis_index("c")
        base = wid * bpw
        pltpu.sync_copy(idx_hbm.at[pl.ds(base, bpw)], idx_v)
        pltpu.async_copy(cb_hbm.at[idx_v], rows_v, sem).wait()
        pltpu.sync_copy(rows_v, out_hbm.at[pl.ds(base, bpw)])

    qrows = _sc_gather(codebook, codes.reshape(BT))

    quant, loss_sum = pl.pallas_call(
        _epilogue_kernel,
        grid=(1,),
        in_specs=[
            pl.BlockSpec((B, T, C), lambda i: (0, 0, 0)),
            pl.BlockSpec((B, C, T), lambda i: (0, 0, 0)),
        ],
        out_specs=[
            pl.BlockSpec((B, C, T), lambda i: (0, 0, 0)),
            pl.BlockSpec((1, 1), lambda i: (0, 0)),
        ],
        out_shape=[
            jax.ShapeDtypeStruct((B, C, T), jnp.float32),
            jax.ShapeDtypeStruct((1, 1), jnp.float32),
        ],
    )(qrows.reshape(B, T, C), z)

    commit_loss = loss_sum[0, 0] / (B * C * T)
    return codes, quant, commit_loss
